# Initial kernel scaffold; baseline (speedup 1.0000x reference)
import jax
import jax.numpy as jnp
from jax.experimental import pallas as pl


def kernel(x, edge_index, W, a_l, a_r):
    raise NotImplementedError("write your pallas kernel here")


# speedup vs baseline: 14.7184x; 14.7184x over previous
import functools

import jax
import jax.numpy as jnp
from jax import lax
from jax.experimental import pallas as pl
from jax.experimental.pallas import tpu as pltpu
from jax.experimental.pallas import tpu_sc as plsc

_F = 128
_NP = 10240
_TILES = 32
_K = 128
_SUB = 16
_LANES = 16


def _proj_body(x_ref, w_ref, a_ref, h_ref, p_ref):
    h = jnp.dot(x_ref[...], w_ref[...], preferred_element_type=jnp.float32)
    h = jnp.where(jnp.isnan(h), 0.0, h)
    h_ref[...] = h
    p_ref[...] = jnp.dot(h, a_ref[...], preferred_element_type=jnp.float32)


def _project(xp, w, a):
    grid = _NP // _F
    return pl.pallas_call(
        _proj_body,
        grid=(grid,),
        in_specs=[
            pl.BlockSpec((_F, _F), lambda i: (i, 0)),
            pl.BlockSpec((_F, _F), lambda i: (0, 0)),
            pl.BlockSpec((_F, _F), lambda i: (0, 0)),
        ],
        out_specs=[
            pl.BlockSpec((_F, _F), lambda i: (i, 0)),
            pl.BlockSpec((_F, _F), lambda i: (i, 0)),
        ],
        out_shape=[
            jax.ShapeDtypeStruct((_NP, _F), jnp.float32),
            jax.ShapeDtypeStruct((_NP, _F), jnp.float32),
        ],
    )(xp, w, a)


def _comb_body(p_ref, o_ref):
    o_ref[...] = p_ref[0] + p_ref[1]


def _combine(part):
    return pl.pallas_call(
        _comb_body,
        grid=(_NP // _F,),
        in_specs=[pl.BlockSpec((2, _F, _F), lambda i: (0, i, 0))],
        out_specs=pl.BlockSpec((_F, _F), lambda i: (i, 0)),
        out_shape=jax.ShapeDtypeStruct((_NP, _F), jnp.float32),
    )(part)


def _edge_scores(hl, hr, row3, col3, chunks):
    npt = _NP // _SUB
    mesh = plsc.VectorSubcoreMesh(core_axis_name="c", subcore_axis_name="s")

    @functools.partial(
        pl.kernel,
        compiler_params=pltpu.CompilerParams(needs_layout_passes=False),
        out_type=[
            jax.ShapeDtypeStruct((_TILES, chunks, _K), jnp.float32),
            jax.ShapeDtypeStruct((2, _NP), jnp.float32),
        ],
        mesh=mesh,
        scratch_types=[
            pltpu.VMEM((_NP,), jnp.float32),
            pltpu.VMEM((_NP,), jnp.float32),
            pltpu.VMEM((chunks, _K), jnp.int32),
            pltpu.VMEM((chunks, _K), jnp.int32),
            pltpu.VMEM((chunks, _K), jnp.float32),
            pltpu.VMEM((_NP,), jnp.float32),
            pltpu.VMEM((_NP,), jnp.float32),
            pltpu.VMEM_SHARED((_SUB, _NP), jnp.float32),
        ],
    )
    def k(hl_h, hr_h, row_h, col_h, ee_h, dp_h,
          hl_v, hr_v, row_v, col_v, ee_v, den_v, tmp_v, stage):
        c = lax.axis_index("c")
        s = lax.axis_index("s")
        wid = c * _SUB + s
        pltpu.sync_copy(hl_h, hl_v)
        pltpu.sync_copy(hr_h, hr_v)
        pltpu.sync_copy(row_h.at[wid], row_v)
        pltpu.sync_copy(col_h.at[wid], col_v)

        zero16 = jnp.zeros((_LANES,), jnp.float32)

        def zbody(i, _):
            den_v[pl.ds(i * _LANES, _LANES)] = zero16
            return 0
        lax.fori_loop(0, _NP // _LANES, zbody, 0)

        vecs_per_chunk = _K // _LANES

        def ebody(i, _):
            j = i // vecs_per_chunk
            m = i % vecs_per_chunk
            rv = row_v[j, pl.ds(m * _LANES, _LANES)]
            cv = col_v[j, pl.ds(m * _LANES, _LANES)]
            e = plsc.load_gather(hl_v, [rv]) + plsc.load_gather(hr_v, [cv])
            e = jnp.where(e > 0, e, 0.2 * e)
            ee = jnp.exp(e)
            ee_v[j, pl.ds(m * _LANES, _LANES)] = ee
            plsc.addupdate_scatter(den_v, [rv], ee)
            return 0
        lax.fori_loop(0, chunks * vecs_per_chunk, ebody, 0)

        pltpu.sync_copy(ee_v, ee_h.at[wid])

        pltpu.sync_copy(den_v, stage.at[s])
        plsc.subcore_barrier()
        base = s * npt
        pltpu.sync_copy(stage.at[0, pl.ds(base, npt)], tmp_v.at[pl.ds(0, npt)])
        for t in range(1, _SUB):
            pltpu.sync_copy(stage.at[t, pl.ds(base, npt)],
                            den_v.at[pl.ds(0, npt)])

            def abody(i, _):
                sl = pl.ds(i * _LANES, _LANES)
                tmp_v[sl] = tmp_v[sl] + den_v[sl]
                return 0
            lax.fori_loop(0, npt // _LANES, abody, 0)
        pltpu.sync_copy(tmp_v.at[pl.ds(0, npt)], dp_h.at[c, pl.ds(base, npt)])

    return k(hl, hr, row3, col3)


def _aggregate(h, ee3, dpart, row3, col3, chunks):
    npt = _NP // _SUB
    mesh = plsc.VectorSubcoreMesh(core_axis_name="c", subcore_axis_name="s")

    @functools.partial(
        pl.kernel,
        compiler_params=pltpu.CompilerParams(needs_layout_passes=False),
        out_type=jax.ShapeDtypeStruct((2, _NP, _F), jnp.float32),
        mesh=mesh,
        scratch_types=[
            pltpu.VMEM((_NP,), jnp.float32),
            pltpu.VMEM((_NP,), jnp.float32),
            pltpu.VMEM((_K,), jnp.int32),
            pltpu.VMEM((_K,), jnp.int32),
            pltpu.VMEM((_K,), jnp.float32),
            pltpu.VMEM((_K, _F), jnp.float32),
            pltpu.VMEM((_LANES,), jnp.float32),
            pltpu.VMEM_SHARED((_NP, _F), jnp.float32),
        ],
    )
    def k(h_h, ee_h, dp_h, row_h, col_h, out_h,
          den_v, den2_v, rowc, colc, eec, rows_v, att_v, acc):
        c = lax.axis_index("c")
        s = lax.axis_index("s")
        wid = c * _SUB + s
        pltpu.sync_copy(dp_h.at[0], den_v)
        pltpu.sync_copy(dp_h.at[1], den2_v)

        def dbody(i, _):
            sl = pl.ds(i * _LANES, _LANES)
            den_v[sl] = den_v[sl] + den2_v[sl]
            return 0
        lax.fori_loop(0, _NP // _LANES, dbody, 0)

        zero16 = jnp.zeros((_LANES,), jnp.float32)

        def zbody(i, _):
            r = i // (_F // _LANES)
            q = i % (_F // _LANES)
            rows_v[r, pl.ds(q * _LANES, _LANES)] = zero16
            return 0
        lax.fori_loop(0, _K * (_F // _LANES), zbody, 0)
        for t in range(npt // _K):
            pltpu.sync_copy(rows_v, acc.at[pl.ds(s * npt + t * _K, _K)])
        plsc.subcore_barrier()

        vecs_per_chunk = _K // _LANES

        def chunk(j, _):
            pltpu.sync_copy(row_h.at[wid, j], rowc)
            pltpu.sync_copy(col_h.at[wid, j], colc)
            pltpu.sync_copy(ee_h.at[wid, j], eec)
            pltpu.sync_copy(h_h.at[colc], rows_v)
            for m in range(vecs_per_chunk):
                rv = rowc[pl.ds(m * _LANES, _LANES)]
                dg = plsc.load_gather(den_v, [rv])
                eev = eec[pl.ds(m * _LANES, _LANES)]
                att_v[...] = eev / dg

                def rbody(r, _):
                    ab = plsc.load_gather(
                        att_v, [jnp.broadcast_to(r, (_LANES,))])
                    ri = m * _LANES + r
                    for q in range(_F // _LANES):
                        sl = pl.ds(q * _LANES, _LANES)
                        rows_v[ri, sl] = rows_v[ri, sl] * ab
                    return 0
                lax.fori_loop(0, _LANES, rbody, 0)
            pltpu.sync_copy(rows_v, acc.at[rowc], add=True)
            return 0
        lax.fori_loop(0, chunks, chunk, 0)

        plsc.subcore_barrier()
        for t in range(npt // _K):
            off = s * npt + t * _K
            pltpu.sync_copy(acc.at[pl.ds(off, _K)], rows_v)
            pltpu.sync_copy(rows_v, out_h.at[c, pl.ds(off, _K)])

    return k(h, ee3, dpart, row3, col3)


def kernel(x, edge_index, W, a_l, a_r):
    n, f_in = x.shape
    f_out = W.shape[1]
    e = edge_index.shape[1]

    xp = jnp.pad(x, ((0, _NP - n), (0, 0)))
    a_cols = jnp.zeros((f_out, _F), jnp.float32)
    a_cols = a_cols.at[:, 0].set(a_l.reshape(-1))
    a_cols = a_cols.at[:, 1].set(a_r.reshape(-1))

    ep = _TILES * _K * ((e + _TILES * _K - 1) // (_TILES * _K))
    chunks = ep // (_TILES * _K)
    row = edge_index[0]
    col = edge_index[1]
    row3 = jnp.concatenate(
        [row, jnp.full((ep - e,), n, jnp.int32)]).reshape(_TILES, chunks, _K)
    col3 = jnp.concatenate(
        [col, jnp.zeros((ep - e,), jnp.int32)]).reshape(_TILES, chunks, _K)

    h, proj = _project(xp, W, a_cols)
    hl = proj[:, 0]
    hr = proj[:, 1]

    ee3, dpart = _edge_scores(hl, hr, row3, col3, chunks)
    part = _aggregate(h, ee3, dpart, row3, col3, chunks)
    out = _combine(part)
    return out[:n]
